# Initial kernel scaffold; baseline (speedup 1.0000x reference)
import functools

import jax
import jax.numpy as jnp
from jax import lax
from jax.experimental import pallas as pl
from jax.experimental.pallas import tpu as pltpu
from jax.experimental.pallas import tpu_sc as plsc

N_NODES = 10000
N_EDGES = 320000

NC, NS, L = 2, 16, 16
NTILES = NC * NS
BW = 313
MAGIC = 13401
NP = NTILES * BW
CAP = 321024
FLUSH = 512
PCHUNK = 4000
EC = 32
SBLK = 1024

_SC_PARAMS = pltpu.CompilerParams(needs_layout_passes=False)


def _mesh():
    return plsc.VectorSubcoreMesh(core_axis_name="c", subcore_axis_name="s")


@functools.partial(
    pl.kernel,
    out_type=[
        jax.ShapeDtypeStruct((NTILES * CAP,), jnp.int32),
        jax.ShapeDtypeStruct((NTILES * CAP,), jnp.int32),
        jax.ShapeDtypeStruct((NTILES, 16), jnp.int32),
    ],
    mesh=_mesh(),
    scratch_types=[
        pltpu.VMEM((PCHUNK,), jnp.int32),
        pltpu.VMEM((PCHUNK,), jnp.int32),
        pltpu.VMEM((FLUSH + L,), jnp.int32),
        pltpu.VMEM((FLUSH + L,), jnp.int32),
        pltpu.VMEM((16,), jnp.int32),
    ],
    compiler_params=_SC_PARAMS,
)
def _bucket_edges(src_hbm, dst_hbm, bsrc_hbm, bdst_hbm, cnt_hbm,
                  s_stage, d_stage, s_buf, d_buf, cnt_v):
    wid = lax.axis_index("s") * NC + lax.axis_index("c")
    base = wid * CAP
    zero16 = jnp.zeros((L,), jnp.int32)
    for k in range((FLUSH + L) // L):
        s_buf[pl.ds(k * L, L)] = zero16
        d_buf[pl.ds(k * L, L)] = zero16

    def chunk_body(ch, carry):
        fill, written = carry
        pltpu.sync_copy(src_hbm.at[pl.ds(ch * PCHUNK, PCHUNK)], s_stage)
        pltpu.sync_copy(dst_hbm.at[pl.ds(ch * PCHUNK, PCHUNK)], d_stage)

        def grp_body(g, carry2):
            fill, written = carry2
            vs = s_stage[pl.ds(g * L, L)]
            vd = d_stage[pl.ds(g * L, L)]
            m = ((vd * jnp.int32(MAGIC)) >> jnp.int32(22)) == wid
            plsc.store_compressed(s_buf.at[pl.ds(fill, L)], vs, mask=m)
            plsc.store_compressed(d_buf.at[pl.ds(fill, L)], vd, mask=m)
            fill = fill + plsc.all_reduce_population_count(m)[0]

            def do_flush(args):
                fill, written = args
                pltpu.sync_copy(s_buf.at[pl.ds(0, FLUSH)],
                                bsrc_hbm.at[pl.ds(base + written, FLUSH)])
                pltpu.sync_copy(d_buf.at[pl.ds(0, FLUSH)],
                                bdst_hbm.at[pl.ds(base + written, FLUSH)])
                ts = s_buf[pl.ds(FLUSH, L)]
                td = d_buf[pl.ds(FLUSH, L)]
                s_buf[pl.ds(0, L)] = ts
                d_buf[pl.ds(0, L)] = td
                return fill - FLUSH, written + FLUSH

            return lax.cond(fill >= FLUSH, do_flush,
                            lambda a: a, (fill, written))

        return lax.fori_loop(0, PCHUNK // L, grp_body, (fill, written))

    fill, written = lax.fori_loop(0, N_EDGES // PCHUNK, chunk_body,
                                  (jnp.int32(0), jnp.int32(0)))
    pltpu.sync_copy(s_buf, bsrc_hbm.at[pl.ds(base + written, FLUSH + L)])
    pltpu.sync_copy(d_buf, bdst_hbm.at[pl.ds(base + written, FLUSH + L)])
    cnt_v[...] = lax.broadcast(written + fill, (L,))
    pltpu.sync_copy(cnt_v, cnt_hbm.at[wid])


def _make_edge_kernel(hoc, heads):
    oc = hoc // heads
    kv = hoc // L
    kvh = oc // L

    @functools.partial(
        pl.kernel,
        out_type=[
            jax.ShapeDtypeStruct((NP, hoc), jnp.float32),
            jax.ShapeDtypeStruct((NP, heads * L), jnp.float32),
        ],
        mesh=_mesh(),
        scratch_types=[
            pltpu.VMEM((SBLK,), jnp.int32),
            pltpu.VMEM((SBLK,), jnp.int32),
            pltpu.VMEM((EC, hoc), jnp.float32),
            pltpu.VMEM((EC, hoc), jnp.float32),
            pltpu.VMEM((BW, hoc), jnp.float32),
            pltpu.VMEM((BW, heads * L), jnp.float32),
            pltpu.VMEM((hoc,), jnp.float32),
            pltpu.VMEM((16,), jnp.int32),
            pltpu.SemaphoreType.DMA,
            pltpu.SemaphoreType.DMA,
        ],
        compiler_params=_SC_PARAMS,
    )
    def edge_kernel(xl_hbm, xr_hbm, bsrc_hbm, bdst_hbm, cnt_hbm, att_hbm,
                    acc_hbm, den_hbm,
                    sidx_v, didx_v, xl_v, xr_v, acc_v, den_v, att_v, cnt_v,
                    sem0, sem1):
        wid = lax.axis_index("s") * NC + lax.axis_index("c")
        base = wid * CAP
        nbase = wid * BW

        pltpu.sync_copy(cnt_hbm.at[wid], cnt_v)
        n = cnt_v[...][0]
        pltpu.sync_copy(att_hbm, att_v)
        att = [att_v[pl.ds(k * L, L)] for k in range(kv)]

        zf = jnp.zeros((L,), jnp.float32)

        def zero_body(r, _):
            for k in range(kv):
                acc_v[r, pl.ds(k * L, L)] = zf
            for h in range(heads):
                den_v[r, pl.ds(h * L, L)] = zf
            return 0

        lax.fori_loop(0, BW, zero_body, 0)

        ntrips = (n + EC - 1) // EC

        def chunk_body(c, _):
            off = (c & jnp.int32(SBLK // EC - 1)) * EC

            @pl.when(off == 0)
            def _():
                pltpu.sync_copy(bsrc_hbm.at[pl.ds(base + c * EC, SBLK)], sidx_v)
                pltpu.sync_copy(bdst_hbm.at[pl.ds(base + c * EC, SBLK)], didx_v)

            cp0 = pltpu.async_copy(xl_hbm.at[sidx_v.at[pl.ds(off, EC)]], xl_v, sem0)
            cp1 = pltpu.async_copy(xr_hbm.at[didx_v.at[pl.ds(off, EC)]], xr_v, sem1)
            cp0.wait()
            cp1.wait()

            for g in range(EC // L):
                dst16 = didx_v[pl.ds(off + g * L, L)]
                for i in range(L):
                    j = g * L + i
                    ej = c * EC + j
                    valid = ej < n
                    row = jnp.where(valid, dst16[i] - nbase, 0)
                    gate = lax.broadcast(
                        jnp.where(valid, jnp.float32(1.0), jnp.float32(0.0)),
                        (L,))
                    xlr = [xl_v[j, pl.ds(k * L, L)] for k in range(kv)]
                    ps = []
                    for h in range(heads):
                        asum = zf
                        for kk in range(kvh):
                            k = h * kvh + kk
                            t = xlr[k] + xr_v[j, pl.ds(k * L, L)]
                            lr = 0.6 * t + 0.4 * jnp.abs(t)
                            asum = asum + lr * att[k]
                        s = jnp.sum(asum)
                        ps.append(jnp.exp(lax.broadcast(s, (L,))) * gate)
                    for h in range(heads):
                        plsc.addupdate(den_v.at[row, pl.ds(h * L, L)], ps[h])
                        for kk in range(kvh):
                            k = h * kvh + kk
                            plsc.addupdate(acc_v.at[row, pl.ds(k * L, L)],
                                           xlr[k] * ps[h])
            return 0

        lax.fori_loop(0, ntrips, chunk_body, 0)
        pltpu.sync_copy(acc_v, acc_hbm.at[pl.ds(nbase, BW)])
        pltpu.sync_copy(den_v, den_hbm.at[pl.ds(nbase, BW)])

    return edge_kernel


_edge_kernel_h2 = _make_edge_kernel(256, 2)
_edge_kernel_h1 = _make_edge_kernel(128, 1)


def _mm_first(x, wl, wr):
    n, din = x.shape
    hoc = wl.shape[1]
    blk = 1000

    def body(x_ref, wl_ref, wr_ref, xl_ref, xr_ref):
        h = x_ref[...]
        xl_ref[...] = jnp.dot(h, wl_ref[...], preferred_element_type=jnp.float32)
        xr_ref[...] = jnp.dot(h, wr_ref[...], preferred_element_type=jnp.float32)

    return pl.pallas_call(
        body,
        grid=(n // blk,),
        in_specs=[
            pl.BlockSpec((blk, din), lambda i: (i, 0)),
            pl.BlockSpec((din, hoc), lambda i: (0, 0)),
            pl.BlockSpec((din, hoc), lambda i: (0, 0)),
        ],
        out_specs=[
            pl.BlockSpec((blk, hoc), lambda i: (i, 0)),
            pl.BlockSpec((blk, hoc), lambda i: (i, 0)),
        ],
        out_shape=[
            jax.ShapeDtypeStruct((n, hoc), jnp.float32),
            jax.ShapeDtypeStruct((n, hoc), jnp.float32),
        ],
    )(x, wl, wr)


def _mm_fused(acc, den, bias, wl, wr, heads_prev):
    n, din = acc.shape
    hoc = wl.shape[1]
    ocp = din // heads_prev
    blk = 2504
    grid = n // blk

    def body(acc_ref, den_ref, b_ref, wl_ref, wr_ref, xl_ref, xr_ref):
        a = acc_ref[...]
        d = den_ref[...]
        parts = [
            jnp.broadcast_to(d[:, h * L:h * L + 1], (blk, ocp))
            for h in range(heads_prev)
        ]
        div = jnp.concatenate(parts, axis=1)
        h = jnp.maximum(a / (div + 1e-16) + b_ref[...], 0.0)
        xl_ref[...] = jnp.dot(h, wl_ref[...], preferred_element_type=jnp.float32)
        xr_ref[...] = jnp.dot(h, wr_ref[...], preferred_element_type=jnp.float32)

    return pl.pallas_call(
        body,
        grid=(grid,),
        in_specs=[
            pl.BlockSpec((blk, din), lambda i: (i, 0)),
            pl.BlockSpec((blk, heads_prev * L), lambda i: (i, 0)),
            pl.BlockSpec((1, din), lambda i: (0, 0)),
            pl.BlockSpec((din, hoc), lambda i: (0, 0)),
            pl.BlockSpec((din, hoc), lambda i: (0, 0)),
        ],
        out_specs=[
            pl.BlockSpec((blk, hoc), lambda i: (i, 0)),
            pl.BlockSpec((blk, hoc), lambda i: (i, 0)),
        ],
        out_shape=[
            jax.ShapeDtypeStruct((n, hoc), jnp.float32),
            jax.ShapeDtypeStruct((n, hoc), jnp.float32),
        ],
    )(acc, den, bias, wl, wr)


def _finalize(acc, den, bias):
    hoc = acc.shape[1]
    blk = 1000

    def body(acc_ref, den_ref, b_ref, out_ref):
        a = acc_ref[...]
        d = jnp.broadcast_to(den_ref[:, 0:1], (blk, hoc))
        out_ref[...] = a / (d + 1e-16) + b_ref[...]

    return pl.pallas_call(
        body,
        grid=(N_NODES // blk,),
        in_specs=[
            pl.BlockSpec((blk, hoc), lambda i: (i, 0)),
            pl.BlockSpec((blk, L), lambda i: (i, 0)),
            pl.BlockSpec((1, hoc), lambda i: (0, 0)),
        ],
        out_specs=pl.BlockSpec((blk, hoc), lambda i: (i, 0)),
        out_shape=jax.ShapeDtypeStruct((N_NODES, hoc), jnp.float32),
    )(acc, den, bias)


def kernel(x, edge_index, Wl1, Wr1, att1, b1, Wl2, Wr2, att2, b2,
           Wl3, Wr3, att3, b3, Wl4, Wr4, att4, b4):
    src = edge_index[0]
    dst = edge_index[1]
    bsrc, bdst, cnts = _bucket_edges(src, dst)

    xl, xr = _mm_first(x, Wl1, Wr1)
    acc, den = _edge_kernel_h2(xl, xr, bsrc, bdst, cnts, att1.reshape(-1))

    xl, xr = _mm_fused(acc, den, b1.reshape(1, -1), Wl2, Wr2, 2)
    acc, den = _edge_kernel_h2(xl, xr, bsrc, bdst, cnts, att2.reshape(-1))

    xl, xr = _mm_fused(acc, den, b2.reshape(1, -1), Wl3, Wr3, 2)
    acc, den = _edge_kernel_h2(xl, xr, bsrc, bdst, cnts, att3.reshape(-1))

    xl, xr = _mm_fused(acc, den, b3.reshape(1, -1), Wl4, Wr4, 2)
    acc, den = _edge_kernel_h1(xl, xr, bsrc, bdst, cnts, att4.reshape(-1))

    return _finalize(acc, den, b4.reshape(1, -1))


# speedup vs baseline: 13.1166x; 13.1166x over previous
import functools

import jax
import jax.numpy as jnp
from jax import lax
from jax.experimental import pallas as pl
from jax.experimental.pallas import tpu as pltpu
from jax.experimental.pallas import tpu_sc as plsc

N_NODES = 10000
N_EDGES = 320000

NC, NS, L = 2, 16, 16
NTILES = NC * NS
BW = 320
MAGIC = 6554
MSHIFT = 21
NP = NTILES * BW
CAP = 321024
FLUSH = 512
PCHUNK = 4000
EC = 32
SBLK = 512

_SC_PARAMS = pltpu.CompilerParams(needs_layout_passes=False)


def _mesh():
    return plsc.VectorSubcoreMesh(core_axis_name="c", subcore_axis_name="s",
                                  num_cores=NC, num_subcores=NS)


@functools.partial(
    pl.kernel,
    out_type=[
        jax.ShapeDtypeStruct((NTILES * CAP,), jnp.int32),
        jax.ShapeDtypeStruct((NTILES * CAP,), jnp.int32),
        jax.ShapeDtypeStruct((NTILES, 16), jnp.int32),
    ],
    mesh=_mesh(),
    scratch_types=[
        pltpu.VMEM((PCHUNK,), jnp.int32),
        pltpu.VMEM((PCHUNK,), jnp.int32),
        pltpu.VMEM((FLUSH + L,), jnp.int32),
        pltpu.VMEM((FLUSH + L,), jnp.int32),
        pltpu.VMEM((16,), jnp.int32),
    ],
    compiler_params=_SC_PARAMS,
)
def _bucket_edges(src_hbm, dst_hbm, bsrc_hbm, bdst_hbm, cnt_hbm,
                  s_stage, d_stage, s_buf, d_buf, cnt_v):
    wid = lax.axis_index("s") * NC + lax.axis_index("c")
    base = wid * CAP
    zero16 = jnp.zeros((L,), jnp.int32)
    for k in range((FLUSH + L) // L):
        s_buf[pl.ds(k * L, L)] = zero16
        d_buf[pl.ds(k * L, L)] = zero16

    def chunk_body(ch, carry):
        fill, written = carry
        coff = pl.multiple_of(ch * PCHUNK, 8)
        pltpu.sync_copy(src_hbm.at[pl.ds(coff, PCHUNK)], s_stage)
        pltpu.sync_copy(dst_hbm.at[pl.ds(coff, PCHUNK)], d_stage)

        def grp_body(g, carry2):
            fill, written = carry2
            vs = s_stage[pl.ds(g * L, L)]
            vd = d_stage[pl.ds(g * L, L)]
            m = ((vd * jnp.int32(MAGIC)) >> jnp.int32(MSHIFT)) == wid
            plsc.store_compressed(s_buf.at[pl.ds(fill, L)], vs, mask=m)
            plsc.store_compressed(d_buf.at[pl.ds(fill, L)], vd, mask=m)
            fill = fill + plsc.all_reduce_population_count(m)[0]

            need_flush = fill >= FLUSH

            @pl.when(need_flush)
            def _():
                woff = pl.multiple_of(base + written, 8)
                pltpu.sync_copy(s_buf.at[pl.ds(0, FLUSH)],
                                bsrc_hbm.at[pl.ds(woff, FLUSH)])
                pltpu.sync_copy(d_buf.at[pl.ds(0, FLUSH)],
                                bdst_hbm.at[pl.ds(woff, FLUSH)])
                ts = s_buf[pl.ds(FLUSH, L)]
                td = d_buf[pl.ds(FLUSH, L)]
                s_buf[pl.ds(0, L)] = ts
                d_buf[pl.ds(0, L)] = td

            adj = jnp.where(need_flush, jnp.int32(FLUSH), jnp.int32(0))
            return fill - adj, written + adj

        return lax.fori_loop(0, PCHUNK // L, grp_body, (fill, written))

    fill, written = lax.fori_loop(0, N_EDGES // PCHUNK, chunk_body,
                                  (jnp.int32(0), jnp.int32(0)))
    woff = pl.multiple_of(base + written, 8)
    pltpu.sync_copy(s_buf, bsrc_hbm.at[pl.ds(woff, FLUSH + L)])
    pltpu.sync_copy(d_buf, bdst_hbm.at[pl.ds(woff, FLUSH + L)])
    cnt_v[...] = lax.broadcast(written + fill, (L,))
    pltpu.sync_copy(cnt_v, cnt_hbm.at[wid])


def _make_edge_kernel(hoc, heads):
    oc = hoc // heads
    kv = hoc // L
    kvh = oc // L

    @functools.partial(
        pl.kernel,
        out_type=[
            jax.ShapeDtypeStruct((NP, hoc), jnp.float32),
            jax.ShapeDtypeStruct((NP * heads * L,), jnp.float32),
        ],
        mesh=_mesh(),
        scratch_types=[
            pltpu.VMEM((SBLK,), jnp.int32),
            pltpu.VMEM((SBLK,), jnp.int32),
            pltpu.VMEM((2, EC, hoc), jnp.float32),
            pltpu.VMEM((2, EC, hoc), jnp.float32),
            pltpu.VMEM((BW, hoc), jnp.float32),
            pltpu.VMEM((BW * heads * L,), jnp.float32),
            pltpu.VMEM((hoc,), jnp.float32),
            pltpu.VMEM((16,), jnp.int32),
            pltpu.SemaphoreType.DMA,
            pltpu.SemaphoreType.DMA,
        ],
        compiler_params=_SC_PARAMS,
    )
    def edge_kernel(xl_hbm, xr_hbm, bsrc_hbm, bdst_hbm, cnt_hbm, att_hbm,
                    acc_hbm, den_hbm,
                    sidx_v, didx_v, xl_v, xr_v, acc_v, den_v, att_v, cnt_v,
                    sem0, sem1):
        wid = lax.axis_index("s") * NC + lax.axis_index("c")
        base = wid * CAP
        nbase = wid * BW

        pltpu.sync_copy(cnt_hbm.at[wid], cnt_v)
        n = jnp.minimum(cnt_v[...][0], jnp.int32(CAP - SBLK))
        pltpu.sync_copy(att_hbm, att_v)
        att = [att_v[pl.ds(k * L, L)] for k in range(kv)]

        zf = jnp.zeros((L,), jnp.float32)

        def zero_body(r, _):
            for k in range(kv):
                acc_v[r, pl.ds(k * L, L)] = zf
            for h in range(heads):
                den_v[pl.ds(r * (heads * L) + h * L, L)] = zf
            return 0

        lax.fori_loop(0, BW, zero_body, 0)

        ntrips = (n + jnp.int32(EC - 1)) >> jnp.int32(5)
        SBMASK = jnp.int32(SBLK // EC - 1)

        def stage(c):
            soff = pl.multiple_of(base + c * EC, 8)
            pltpu.sync_copy(bsrc_hbm.at[pl.ds(soff, SBLK)], sidx_v)
            pltpu.sync_copy(bdst_hbm.at[pl.ds(soff, SBLK)], didx_v)

        def issue(c, slot, sem):
            off = (c & SBMASK) * EC
            for half in range(EC // L):
                s16 = sidx_v[pl.ds(off + half * L, L)]
                d16 = didx_v[pl.ds(off + half * L, L)]
                pltpu.async_copy(xl_hbm.at[s16],
                                 xl_v.at[slot, pl.ds(half * L, L)], sem)
                pltpu.async_copy(xr_hbm.at[d16],
                                 xr_v.at[slot, pl.ds(half * L, L)], sem)

        def drain(slot, sem):
            for half in range(EC // L):
                pltpu.make_async_copy(
                    xl_hbm.at[pl.ds(0, L)],
                    xl_v.at[slot, pl.ds(half * L, L)], sem).wait()
                pltpu.make_async_copy(
                    xr_hbm.at[pl.ds(0, L)],
                    xr_v.at[slot, pl.ds(half * L, L)], sem).wait()

        @pl.when(ntrips > 0)
        def _():
            stage(0)
            issue(0, 0, sem0)

        def chunk_body(c, _):
            off = (c & SBMASK) * EC
            dst_a = didx_v[pl.ds(off, L)]
            dst_b = didx_v[pl.ds(off + L, L)]
            cn = c + 1

            @pl.when(cn < ntrips)
            def _():
                @pl.when((cn & SBMASK) == 0)
                def _():
                    stage(cn)

                @pl.when((cn & 1) == 0)
                def _():
                    issue(cn, 0, sem0)

                @pl.when((cn & 1) == 1)
                def _():
                    issue(cn, 1, sem1)

            @pl.when((c & 1) == 0)
            def _():
                drain(0, sem0)

            @pl.when((c & 1) == 1)
            def _():
                drain(1, sem1)

            b = c & 1
            for g, dst16 in ((0, dst_a), (1, dst_b)):
                for i in range(L):
                    j = g * L + i
                    ej = c * EC + j
                    valid = ej < n
                    row = jnp.where(valid, dst16[i] - nbase, 0)
                    row = jnp.maximum(jnp.minimum(row, jnp.int32(BW - 1)),
                                      jnp.int32(0))
                    gate = lax.broadcast(
                        jnp.where(valid, jnp.float32(1.0), jnp.float32(0.0)),
                        (L,))
                    xlr = [xl_v[b, j, pl.ds(k * L, L)] for k in range(kv)]
                    ps = []
                    for h in range(heads):
                        asum = zf
                        for kk in range(kvh):
                            k = h * kvh + kk
                            t = xlr[k] + xr_v[b, j, pl.ds(k * L, L)]
                            lr = 0.6 * t + 0.4 * jnp.abs(t)
                            asum = asum + lr * att[k]
                        s = jnp.sum(asum)
                        ps.append(jnp.exp(lax.broadcast(s, (L,))) * gate)
                    for h in range(heads):
                        plsc.addupdate(
                            den_v.at[pl.ds(row * (heads * L) + h * L, L)], ps[h])
                        for kk in range(kvh):
                            k = h * kvh + kk
                            plsc.addupdate(acc_v.at[row, pl.ds(k * L, L)],
                                           xlr[k] * ps[h])
            return 0

        lax.fori_loop(0, ntrips, chunk_body, 0)
        pltpu.sync_copy(acc_v, acc_hbm.at[pl.ds(nbase, BW)])
        pltpu.sync_copy(
            den_v, den_hbm.at[pl.ds(wid * (BW * heads * L), BW * heads * L)])

    return edge_kernel


_edge_kernel_h2 = _make_edge_kernel(256, 2)
_edge_kernel_h1 = _make_edge_kernel(128, 1)


def _mm_first(x, wl, wr):
    n, din = x.shape
    hoc = wl.shape[1]
    blk = 1000

    def body(x_ref, wl_ref, wr_ref, xl_ref, xr_ref):
        h = x_ref[...]
        xl_ref[...] = jnp.dot(h, wl_ref[...], preferred_element_type=jnp.float32)
        xr_ref[...] = jnp.dot(h, wr_ref[...], preferred_element_type=jnp.float32)

    return pl.pallas_call(
        body,
        grid=(n // blk,),
        in_specs=[
            pl.BlockSpec((blk, din), lambda i: (i, 0)),
            pl.BlockSpec((din, hoc), lambda i: (0, 0)),
            pl.BlockSpec((din, hoc), lambda i: (0, 0)),
        ],
        out_specs=[
            pl.BlockSpec((blk, hoc), lambda i: (i, 0)),
            pl.BlockSpec((blk, hoc), lambda i: (i, 0)),
        ],
        out_shape=[
            jax.ShapeDtypeStruct((n, hoc), jnp.float32),
            jax.ShapeDtypeStruct((n, hoc), jnp.float32),
        ],
    )(x, wl, wr)


def _mm_fused(acc, den, bias, wl, wr, heads_prev):
    n, din = acc.shape
    hoc = wl.shape[1]
    ocp = din // heads_prev
    blk = 2048
    grid = n // blk

    def body(acc_ref, den_ref, b_ref, wl_ref, wr_ref, xl_ref, xr_ref):
        a = acc_ref[...]
        d = den_ref[...]
        parts = [
            jnp.broadcast_to(d[:, h * L:h * L + 1], (blk, ocp))
            for h in range(heads_prev)
        ]
        div = jnp.concatenate(parts, axis=1)
        h = jnp.maximum(a / (div + 1e-16) + b_ref[...], 0.0)
        xl_ref[...] = jnp.dot(h, wl_ref[...], preferred_element_type=jnp.float32)
        xr_ref[...] = jnp.dot(h, wr_ref[...], preferred_element_type=jnp.float32)

    return pl.pallas_call(
        body,
        grid=(grid,),
        in_specs=[
            pl.BlockSpec((blk, din), lambda i: (i, 0)),
            pl.BlockSpec((blk, heads_prev * L), lambda i: (i, 0)),
            pl.BlockSpec((1, din), lambda i: (0, 0)),
            pl.BlockSpec((din, hoc), lambda i: (0, 0)),
            pl.BlockSpec((din, hoc), lambda i: (0, 0)),
        ],
        out_specs=[
            pl.BlockSpec((blk, hoc), lambda i: (i, 0)),
            pl.BlockSpec((blk, hoc), lambda i: (i, 0)),
        ],
        out_shape=[
            jax.ShapeDtypeStruct((n, hoc), jnp.float32),
            jax.ShapeDtypeStruct((n, hoc), jnp.float32),
        ],
    )(acc, den, bias, wl, wr)


def _finalize(acc, den, bias):
    hoc = acc.shape[1]
    blk = 1000

    def body(acc_ref, den_ref, b_ref, out_ref):
        a = acc_ref[...]
        d = jnp.broadcast_to(den_ref[:, 0:1], (blk, hoc))
        out_ref[...] = a / (d + 1e-16) + b_ref[...]

    return pl.pallas_call(
        body,
        grid=(N_NODES // blk,),
        in_specs=[
            pl.BlockSpec((blk, hoc), lambda i: (i, 0)),
            pl.BlockSpec((blk, L), lambda i: (i, 0)),
            pl.BlockSpec((1, hoc), lambda i: (0, 0)),
        ],
        out_specs=pl.BlockSpec((blk, hoc), lambda i: (i, 0)),
        out_shape=jax.ShapeDtypeStruct((N_NODES, hoc), jnp.float32),
    )(acc, den, bias)


def kernel(x, edge_index, Wl1, Wr1, att1, b1, Wl2, Wr2, att2, b2,
           Wl3, Wr3, att3, b3, Wl4, Wr4, att4, b4):
    src = edge_index[0]
    dst = edge_index[1]
    bsrc, bdst, cnts = _bucket_edges(src, dst)

    xl, xr = _mm_first(x, Wl1, Wr1)
    acc, den = _edge_kernel_h2(xl, xr, bsrc, bdst, cnts, att1.reshape(-1))

    xl, xr = _mm_fused(acc, den.reshape(NP, 2 * L), b1.reshape(1, -1), Wl2, Wr2, 2)
    acc, den = _edge_kernel_h2(xl, xr, bsrc, bdst, cnts, att2.reshape(-1))

    xl, xr = _mm_fused(acc, den.reshape(NP, 2 * L), b2.reshape(1, -1), Wl3, Wr3, 2)
    acc, den = _edge_kernel_h2(xl, xr, bsrc, bdst, cnts, att3.reshape(-1))

    xl, xr = _mm_fused(acc, den.reshape(NP, 2 * L), b3.reshape(1, -1), Wl4, Wr4, 2)
    acc, den = _edge_kernel_h1(xl, xr, bsrc, bdst, cnts, att4.reshape(-1))

    return _finalize(acc, den.reshape(NP, L), b4.reshape(1, -1))
